# Initial kernel scaffold; baseline (speedup 1.0000x reference)
import jax
import jax.numpy as jnp
from jax.experimental import pallas as pl


def kernel(x, edge_index, edge_attr, batch, W_proj, b_proj, W_e1, b_e1, W_e2, b_e2, W_root, b_conv, W_gru_ih, b_gru_ih, W_gru_hh, b_gru_hh, W_r1, b_r1, W_r2, b_r2, W_p, b_p):
    raise NotImplementedError("write your pallas kernel here")


# speedup vs baseline: 2.7854x; 2.7854x over previous
import functools

import jax
import jax.numpy as jnp
from jax import lax
from jax.experimental import pallas as pl
from jax.experimental.pallas import tpu as pltpu
from jax.experimental.pallas import tpu_sc as plsc

HID = 8
STEPS = 3
NG = 64
WIN = 128
NC, NS = 2, 16
NW = NC * NS

F32 = jnp.float32


def _tc_proj(x, W, b):
    N, DF = x.shape
    BN = 2000

    def body(x_ref, w_ref, b_ref, o_ref):
        o_ref[...] = jax.nn.relu(
            jnp.dot(x_ref[...], w_ref[...], preferred_element_type=F32)
            + b_ref[...])

    return pl.pallas_call(
        body,
        grid=(N // BN,),
        in_specs=[
            pl.BlockSpec((BN, DF), lambda i: (i, 0)),
            pl.BlockSpec((DF, HID), lambda i: (0, 0)),
            pl.BlockSpec((1, HID), lambda i: (0, 0)),
        ],
        out_specs=pl.BlockSpec((BN, HID), lambda i: (i, 0)),
        out_shape=jax.ShapeDtypeStruct((N, HID), F32),
    )(x, W, b)


def _tc_msg(ea, G, W1, b1, W2, b2):
    E, DE = ea.shape
    BE = 4000
    HH = HID * HID

    def body(ea_ref, g_ref, w1_ref, b1_ref, w2_ref, b2_ref, o_ref):
        u = jax.nn.relu(
            jnp.dot(ea_ref[...], w1_ref[...], preferred_element_type=F32)
            + b1_ref[...])
        wall = jnp.dot(u, w2_ref[...], preferred_element_type=F32) + b2_ref[...]
        ii = lax.broadcasted_iota(jnp.int32, (HID, HH), 0)
        cc = lax.broadcasted_iota(jnp.int32, (HID, HH), 1)
        R = (cc // HID == ii).astype(F32)
        cc2 = lax.broadcasted_iota(jnp.int32, (HH, HID), 0)
        oo = lax.broadcasted_iota(jnp.int32, (HH, HID), 1)
        S = (cc2 % HID == oo).astype(F32)
        grep = jnp.dot(g_ref[...], R, preferred_element_type=F32)
        o_ref[...] = jnp.dot(grep * wall, S, preferred_element_type=F32)

    return pl.pallas_call(
        body,
        grid=(E // BE,),
        in_specs=[
            pl.BlockSpec((BE, DE), lambda i: (i, 0)),
            pl.BlockSpec((BE, HID), lambda i: (i, 0)),
            pl.BlockSpec((DE, DE), lambda i: (0, 0)),
            pl.BlockSpec((1, DE), lambda i: (0, 0)),
            pl.BlockSpec((DE, HH), lambda i: (0, 0)),
            pl.BlockSpec((1, HH), lambda i: (0, 0)),
        ],
        out_specs=pl.BlockSpec((BE, HID), lambda i: (i, 0)),
        out_shape=jax.ShapeDtypeStruct((E, HID), F32),
    )(ea, G, W1, b1, W2, b2)


def _gru_math(p01, h, wr_ref, bc_ref, wih_ref, bih_ref, whh_ref, bhh_ref):
    m = jax.nn.relu(
        p01 + jnp.dot(h, wr_ref[...], preferred_element_type=F32) + bc_ref[...])
    gi = jnp.dot(m, wih_ref[...], preferred_element_type=F32) + bih_ref[...]
    gh = jnp.dot(h, whh_ref[...], preferred_element_type=F32) + bhh_ref[...]
    r = jax.nn.sigmoid(gi[:, 0:HID] + gh[:, 0:HID])
    z = jax.nn.sigmoid(gi[:, HID:2 * HID] + gh[:, HID:2 * HID])
    n = jnp.tanh(gi[:, 2 * HID:3 * HID] + r * gh[:, 2 * HID:3 * HID])
    return (1.0 - z) * n + z * h


def _tc_reduce(p2d):
    NWp, NH = p2d.shape
    BC = 16000

    def body(p_ref, o_ref):
        o_ref[...] = jnp.sum(p_ref[...], axis=0, keepdims=True)

    return pl.pallas_call(
        body,
        grid=(NH // BC,),
        in_specs=[pl.BlockSpec((NWp, BC), lambda i: (0, i))],
        out_specs=pl.BlockSpec((1, BC), lambda i: (0, i)),
        out_shape=jax.ShapeDtypeStruct((1, NH), F32),
    )(p2d)


def _tc_gru(agg, h, Wr, bc, Wih, bih, Whh, bhh):
    N = agg.shape[0]
    BN = 2000

    def body(p_ref, h_ref, wr_ref, bc_ref, wih_ref, bih_ref, whh_ref, bhh_ref,
             o_ref):
        o_ref[...] = _gru_math(p_ref[...], h_ref[...], wr_ref, bc_ref, wih_ref,
                               bih_ref, whh_ref, bhh_ref)

    return pl.pallas_call(
        body,
        grid=(N // BN,),
        in_specs=[
            pl.BlockSpec((BN, HID), lambda i: (i, 0)),
            pl.BlockSpec((BN, HID), lambda i: (i, 0)),
            pl.BlockSpec((HID, HID), lambda i: (0, 0)),
            pl.BlockSpec((1, HID), lambda i: (0, 0)),
            pl.BlockSpec((HID, 3 * HID), lambda i: (0, 0)),
            pl.BlockSpec((1, 3 * HID), lambda i: (0, 0)),
            pl.BlockSpec((HID, 3 * HID), lambda i: (0, 0)),
            pl.BlockSpec((1, 3 * HID), lambda i: (0, 0)),
        ],
        out_specs=pl.BlockSpec((BN, HID), lambda i: (i, 0)),
        out_shape=jax.ShapeDtypeStruct((N, HID), F32),
    )(agg, h, Wr, bc, Wih, bih, Whh, bhh)


def _tc_gru_readout(agg, h, batch3d, Wr, bc, Wih, bih, Whh, bhh,
                    Wr1, br1, Wr2, br2, Wp, bp):
    N = agg.shape[0]
    BN = 2000
    NB = N // BN

    def body(p_ref, h_ref, b_ref, wr_ref, bc_ref, wih_ref, bih_ref, whh_ref,
             bhh_ref, wr1_ref, br1_ref, wr2_ref, br2_ref, wp_ref, bp_ref,
             o_ref, sums, counts):
        i = pl.program_id(0)
        hid = _gru_math(p_ref[...], h_ref[...], wr_ref, bc_ref, wih_ref,
                        bih_ref, whh_ref, bhh_ref)
        nf = jax.nn.relu(
            jnp.dot(hid, wr1_ref[...], preferred_element_type=F32)
            + br1_ref[...])
        w2p = jnp.dot(wr2_ref[...], wp_ref[...], preferred_element_type=F32)
        b2p = jnp.dot(br2_ref[...], wp_ref[...], preferred_element_type=F32)
        t = jnp.dot(nf, w2p, preferred_element_type=F32) + b2p
        b = b_ref[0]
        oh = (lax.broadcasted_iota(jnp.int32, (NG, BN), 0) == b).astype(F32)

        @pl.when(i == 0)
        def _init():
            sums[...] = jnp.zeros((NG, 1), F32)
            counts[...] = jnp.zeros((NG, 1), F32)

        sums[...] += jnp.dot(oh, t, preferred_element_type=F32)
        counts[...] += jnp.sum(oh, axis=1, keepdims=True)

        @pl.when(i == NB - 1)
        def _fin():
            o_ref[...] = (sums[...] / jnp.maximum(counts[...], 1.0)
                          + bp_ref[...])

    return pl.pallas_call(
        body,
        grid=(NB,),
        in_specs=[
            pl.BlockSpec((BN, HID), lambda i: (i, 0)),
            pl.BlockSpec((BN, HID), lambda i: (i, 0)),
            pl.BlockSpec((1, 1, BN), lambda i: (i, 0, 0)),
            pl.BlockSpec((HID, HID), lambda i: (0, 0)),
            pl.BlockSpec((1, HID), lambda i: (0, 0)),
            pl.BlockSpec((HID, 3 * HID), lambda i: (0, 0)),
            pl.BlockSpec((1, 3 * HID), lambda i: (0, 0)),
            pl.BlockSpec((HID, 3 * HID), lambda i: (0, 0)),
            pl.BlockSpec((1, 3 * HID), lambda i: (0, 0)),
            pl.BlockSpec((HID, HID), lambda i: (0, 0)),
            pl.BlockSpec((1, HID), lambda i: (0, 0)),
            pl.BlockSpec((HID, HID), lambda i: (0, 0)),
            pl.BlockSpec((1, HID), lambda i: (0, 0)),
            pl.BlockSpec((HID, 1), lambda i: (0, 0)),
            pl.BlockSpec((1, 1), lambda i: (0, 0)),
        ],
        out_specs=pl.BlockSpec((NG, 1), lambda i: (0, 0)),
        out_shape=jax.ShapeDtypeStruct((NG, 1), F32),
        scratch_shapes=[
            pltpu.VMEM((NG, 1), F32),
            pltpu.VMEM((NG, 1), F32),
        ],
    )(agg, h, batch3d, Wr, bc, Wih, bih, Whh, bhh,
      Wr1, br1, Wr2, br2, Wp, bp)


def _sc_gather(h, src2d):
    R = src2d.shape[0]
    NH = h.shape[0] * HID
    h_flat = h.reshape(NH)
    JMAX = (R + NW - 1) // NW
    mesh = plsc.VectorSubcoreMesh(core_axis_name="c", subcore_axis_name="s",
                                  num_cores=NC, num_subcores=NS)

    @functools.partial(
        pl.kernel,
        out_type=jax.ShapeDtypeStruct((R, WIN * HID), F32),
        mesh=mesh,
        compiler_params=pltpu.CompilerParams(needs_layout_passes=False),
        scratch_types=[
            pltpu.VMEM((NH,), F32),
            pltpu.VMEM((WIN,), jnp.int32),
            pltpu.VMEM((WIN * HID,), F32),
            pltpu.SemaphoreType.DMA,
        ],
    )
    def k(h_hbm, src_hbm, out_hbm, h_v, idx_v, rows_v, sem):
        c = lax.axis_index("c")
        s = lax.axis_index("s")
        wid = s * NC + c
        lane = lax.broadcasted_iota(jnp.int32, (16,), 0)
        pltpu.sync_copy(h_hbm, h_v)

        def body(j, carry):
            row = j * NW + wid

            @pl.when(row < R)
            def _():
                pltpu.sync_copy(src_hbm.at[row], idx_v)
                for g in range(WIN // 16):
                    sidx = idx_v[pl.ds(g * 16, 16)] * HID
                    dbase = (lane + (g * 16)) * HID
                    for o in range(HID):
                        vals = plsc.load_gather(h_v, [sidx + o])
                        plsc.store_scatter(rows_v, [dbase + o], vals)
                pltpu.sync_copy(rows_v, out_hbm.at[row])

            return carry

        lax.fori_loop(0, JMAX, body, 0)

    return k(h_flat, src2d)


def _sc_scatter(msg2d, dst2d, zeros_flat):
    R = msg2d.shape[0]
    NH = zeros_flat.shape[0]
    JMAX = (R + NW - 1) // NW
    mesh = plsc.VectorSubcoreMesh(core_axis_name="c", subcore_axis_name="s",
                                  num_cores=NC, num_subcores=NS)

    @functools.partial(
        pl.kernel,
        out_type=jax.ShapeDtypeStruct((NW, NH), F32),
        mesh=mesh,
        compiler_params=pltpu.CompilerParams(needs_layout_passes=False),
        scratch_types=[
            pltpu.VMEM((NH,), F32),
            pltpu.VMEM((WIN,), jnp.int32),
            pltpu.VMEM((WIN * HID,), F32),
            pltpu.SemaphoreType.DMA,
        ],
    )
    def k(msg_hbm, dst_hbm, zero_hbm, out_hbm, agg_v, idx_v, upd_v, sem):
        c = lax.axis_index("c")
        s = lax.axis_index("s")
        wid = s * NC + c
        lane = lax.broadcasted_iota(jnp.int32, (16,), 0)
        pltpu.sync_copy(zero_hbm, agg_v)

        def body(j, carry):
            row = j * NW + wid

            @pl.when(row < R)
            def _():
                pltpu.sync_copy(dst_hbm.at[row], idx_v)
                pltpu.sync_copy(msg_hbm.at[row], upd_v)
                for g in range(WIN // 16):
                    didx = idx_v[pl.ds(g * 16, 16)] * HID
                    ubase = (lane + (g * 16)) * HID
                    for o in range(HID):
                        vals = plsc.load_gather(upd_v, [ubase + o])
                        plsc.addupdate_scatter(agg_v, [didx + o], vals)

            return carry

        lax.fori_loop(0, JMAX, body, 0)
        pltpu.sync_copy(agg_v, out_hbm.at[wid])

    return k(msg2d, dst2d, zeros_flat)


def kernel(x, edge_index, edge_attr, batch, W_proj, b_proj, W_e1, b_e1, W_e2,
           b_e2, W_root, b_conv, W_gru_ih, b_gru_ih, W_gru_hh, b_gru_hh, W_r1,
           b_r1, W_r2, b_r2, W_p, b_p):
    N = x.shape[0]
    E = edge_index.shape[1]
    R = E // WIN

    src2d = edge_index[0].reshape(R, WIN)
    dst2d = edge_index[1].reshape(R, WIN)
    zeros_flat = jnp.zeros((N * HID,), F32)
    batch3d = batch.reshape(N // 2000, 1, 2000)

    r2 = lambda v: v.reshape(1, -1)
    b_proj2, b_e12, b_e22 = r2(b_proj), r2(b_e1), r2(b_e2)
    b_conv2, b_ih2, b_hh2 = r2(b_conv), r2(b_gru_ih), r2(b_gru_hh)
    br12, br22, bp2 = r2(b_r1), r2(b_r2), r2(b_p)

    h = _tc_proj(x, W_proj, b_proj2)
    G = _sc_gather(h, src2d)
    for step in range(STEPS):
        msg = _tc_msg(edge_attr, G.reshape(E, HID), W_e1, b_e12, W_e2, b_e22)
        partials = _sc_scatter(msg.reshape(R, WIN * HID), dst2d, zeros_flat)
        agg = _tc_reduce(partials).reshape(N, HID)
        if step < STEPS - 1:
            h = _tc_gru(agg, h, W_root, b_conv2, W_gru_ih, b_ih2,
                        W_gru_hh, b_hh2)
            G = _sc_gather(h, src2d)
        else:
            out = _tc_gru_readout(agg, h, batch3d, W_root, b_conv2,
                                  W_gru_ih, b_ih2, W_gru_hh, b_hh2,
                                  W_r1, br12, W_r2, br22, W_p, bp2)
    return out


# speedup vs baseline: 7.1148x; 2.5543x over previous
import functools

import jax
import jax.numpy as jnp
from jax import lax
from jax.experimental import pallas as pl
from jax.experimental.pallas import tpu as pltpu
from jax.experimental.pallas import tpu_sc as plsc

HID = 8
STEPS = 3
NG = 64
WIN = 128
NC, NS = 2, 16
NW = NC * NS

F32 = jnp.float32


def _tc_proj(x, W, b):
    N, DF = x.shape
    BN = 2000

    def body(x_ref, w_ref, b_ref, o_ref):
        o_ref[...] = jax.nn.relu(
            jnp.dot(x_ref[...], w_ref[...], preferred_element_type=F32)
            + b_ref[...])

    return pl.pallas_call(
        body,
        grid=(N // BN,),
        in_specs=[
            pl.BlockSpec((BN, DF), lambda i: (i, 0)),
            pl.BlockSpec((DF, HID), lambda i: (0, 0)),
            pl.BlockSpec((1, HID), lambda i: (0, 0)),
        ],
        out_specs=pl.BlockSpec((BN, HID), lambda i: (i, 0)),
        out_shape=jax.ShapeDtypeStruct((N, HID), F32),
    )(x, W, b)


PK = 8


def _tc_msg(ea8, G8, W1bd, b1bd, W2bd, b2bd, Rbd, Sbd):
    EP, DP = ea8.shape
    HP = G8.shape[1]
    WP = W2bd.shape[1]
    BE = 1000

    def body(ea_ref, g_ref, w1_ref, b1_ref, w2_ref, b2_ref, r_ref, s_ref,
             o_ref):
        u = jax.nn.relu(
            jnp.dot(ea_ref[...], w1_ref[...], preferred_element_type=F32)
            + b1_ref[...])
        wall = jnp.dot(u, w2_ref[...], preferred_element_type=F32) + b2_ref[...]
        grep = jnp.dot(g_ref[...], r_ref[...], preferred_element_type=F32)
        o_ref[...] = jnp.dot(grep * wall, s_ref[...],
                             preferred_element_type=F32)

    return pl.pallas_call(
        body,
        grid=(EP // BE,),
        in_specs=[
            pl.BlockSpec((BE, DP), lambda i: (i, 0)),
            pl.BlockSpec((BE, HP), lambda i: (i, 0)),
            pl.BlockSpec((DP, DP), lambda i: (0, 0)),
            pl.BlockSpec((1, DP), lambda i: (0, 0)),
            pl.BlockSpec((DP, WP), lambda i: (0, 0)),
            pl.BlockSpec((1, WP), lambda i: (0, 0)),
            pl.BlockSpec((HP, WP), lambda i: (0, 0)),
            pl.BlockSpec((WP, HP), lambda i: (0, 0)),
        ],
        out_specs=pl.BlockSpec((BE, HP), lambda i: (i, 0)),
        out_shape=jax.ShapeDtypeStruct((EP, HP), F32),
    )(ea8, G8, W1bd, b1bd, W2bd, b2bd, Rbd, Sbd)


def _gru_math(p01, h, wr_ref, bc_ref, wih_ref, bih_ref, whh_ref, bhh_ref):
    m = jax.nn.relu(
        p01 + jnp.dot(h, wr_ref[...], preferred_element_type=F32) + bc_ref[...])
    gi = jnp.dot(m, wih_ref[...], preferred_element_type=F32) + bih_ref[...]
    gh = jnp.dot(h, whh_ref[...], preferred_element_type=F32) + bhh_ref[...]
    r = jax.nn.sigmoid(gi[:, 0:HID] + gh[:, 0:HID])
    z = jax.nn.sigmoid(gi[:, HID:2 * HID] + gh[:, HID:2 * HID])
    n = jnp.tanh(gi[:, 2 * HID:3 * HID] + r * gh[:, 2 * HID:3 * HID])
    return (1.0 - z) * n + z * h


def _tc_reduce(p2d):
    NWp, NH = p2d.shape
    BC = 16000

    def body(p_ref, o_ref):
        o_ref[...] = jnp.sum(p_ref[...], axis=0, keepdims=True)

    return pl.pallas_call(
        body,
        grid=(NH // BC,),
        in_specs=[pl.BlockSpec((NWp, BC), lambda i: (0, i))],
        out_specs=pl.BlockSpec((1, BC), lambda i: (0, i)),
        out_shape=jax.ShapeDtypeStruct((1, NH), F32),
    )(p2d)


def _tc_gru(agg, h, Wr, bc, Wih, bih, Whh, bhh):
    N = agg.shape[0]
    BN = 2000

    def body(p_ref, h_ref, wr_ref, bc_ref, wih_ref, bih_ref, whh_ref, bhh_ref,
             o_ref):
        o_ref[...] = _gru_math(p_ref[...], h_ref[...], wr_ref, bc_ref, wih_ref,
                               bih_ref, whh_ref, bhh_ref)

    return pl.pallas_call(
        body,
        grid=(N // BN,),
        in_specs=[
            pl.BlockSpec((BN, HID), lambda i: (i, 0)),
            pl.BlockSpec((BN, HID), lambda i: (i, 0)),
            pl.BlockSpec((HID, HID), lambda i: (0, 0)),
            pl.BlockSpec((1, HID), lambda i: (0, 0)),
            pl.BlockSpec((HID, 3 * HID), lambda i: (0, 0)),
            pl.BlockSpec((1, 3 * HID), lambda i: (0, 0)),
            pl.BlockSpec((HID, 3 * HID), lambda i: (0, 0)),
            pl.BlockSpec((1, 3 * HID), lambda i: (0, 0)),
        ],
        out_specs=pl.BlockSpec((BN, HID), lambda i: (i, 0)),
        out_shape=jax.ShapeDtypeStruct((N, HID), F32),
    )(agg, h, Wr, bc, Wih, bih, Whh, bhh)


def _tc_gru_readout(agg, h, batch3d, Wr, bc, Wih, bih, Whh, bhh,
                    Wr1, br1, Wr2, br2, Wp, bp):
    N = agg.shape[0]
    BN = 2000
    NB = N // BN

    def body(p_ref, h_ref, b_ref, wr_ref, bc_ref, wih_ref, bih_ref, whh_ref,
             bhh_ref, wr1_ref, br1_ref, wr2_ref, br2_ref, wp_ref, bp_ref,
             o_ref, sums, counts):
        i = pl.program_id(0)
        hid = _gru_math(p_ref[...], h_ref[...], wr_ref, bc_ref, wih_ref,
                        bih_ref, whh_ref, bhh_ref)
        nf = jax.nn.relu(
            jnp.dot(hid, wr1_ref[...], preferred_element_type=F32)
            + br1_ref[...])
        w2p = jnp.dot(wr2_ref[...], wp_ref[...], preferred_element_type=F32)
        b2p = jnp.dot(br2_ref[...], wp_ref[...], preferred_element_type=F32)
        t = jnp.dot(nf, w2p, preferred_element_type=F32) + b2p
        b = b_ref[0]
        oh = (lax.broadcasted_iota(jnp.int32, (NG, BN), 0) == b).astype(F32)

        @pl.when(i == 0)
        def _init():
            sums[...] = jnp.zeros((NG, 1), F32)
            counts[...] = jnp.zeros((NG, 1), F32)

        sums[...] += jnp.dot(oh, t, preferred_element_type=F32)
        counts[...] += jnp.sum(oh, axis=1, keepdims=True)

        @pl.when(i == NB - 1)
        def _fin():
            o_ref[...] = (sums[...] / jnp.maximum(counts[...], 1.0)
                          + bp_ref[...])

    return pl.pallas_call(
        body,
        grid=(NB,),
        in_specs=[
            pl.BlockSpec((BN, HID), lambda i: (i, 0)),
            pl.BlockSpec((BN, HID), lambda i: (i, 0)),
            pl.BlockSpec((1, 1, BN), lambda i: (i, 0, 0)),
            pl.BlockSpec((HID, HID), lambda i: (0, 0)),
            pl.BlockSpec((1, HID), lambda i: (0, 0)),
            pl.BlockSpec((HID, 3 * HID), lambda i: (0, 0)),
            pl.BlockSpec((1, 3 * HID), lambda i: (0, 0)),
            pl.BlockSpec((HID, 3 * HID), lambda i: (0, 0)),
            pl.BlockSpec((1, 3 * HID), lambda i: (0, 0)),
            pl.BlockSpec((HID, HID), lambda i: (0, 0)),
            pl.BlockSpec((1, HID), lambda i: (0, 0)),
            pl.BlockSpec((HID, HID), lambda i: (0, 0)),
            pl.BlockSpec((1, HID), lambda i: (0, 0)),
            pl.BlockSpec((HID, 1), lambda i: (0, 0)),
            pl.BlockSpec((1, 1), lambda i: (0, 0)),
        ],
        out_specs=pl.BlockSpec((NG, 1), lambda i: (0, 0)),
        out_shape=jax.ShapeDtypeStruct((NG, 1), F32),
        scratch_shapes=[
            pltpu.VMEM((NG, 1), F32),
            pltpu.VMEM((NG, 1), F32),
        ],
    )(agg, h, batch3d, Wr, bc, Wih, bih, Whh, bhh,
      Wr1, br1, Wr2, br2, Wp, bp)


KR = 4


def _sc_gather(h, src2d):
    R = src2d.shape[0]
    NH = h.shape[0] * HID
    h_flat = h.reshape(NH)
    JMAX = (R + NW - 1) // NW
    JO = (JMAX + KR - 1) // KR
    mesh = plsc.VectorSubcoreMesh(core_axis_name="c", subcore_axis_name="s",
                                  num_cores=NC, num_subcores=NS)

    @functools.partial(
        pl.kernel,
        out_type=jax.ShapeDtypeStruct((R, WIN * HID), F32),
        mesh=mesh,
        compiler_params=pltpu.CompilerParams(needs_layout_passes=False),
        scratch_types=[
            pltpu.VMEM((NH,), F32),
            [pltpu.VMEM((WIN,), jnp.int32) for _ in range(KR)],
            [pltpu.VMEM((WIN * HID,), F32) for _ in range(KR)],
            pltpu.SemaphoreType.DMA((KR,)),
            pltpu.SemaphoreType.DMA((KR,)),
        ],
    )
    def k(h_hbm, src_hbm, out_hbm, h_v, idxb, rowsb, isem, osem):
        c_ax = lax.axis_index("c")
        s_ax = lax.axis_index("s")
        wid = s_ax * NC + c_ax
        lane = lax.broadcasted_iota(jnp.int32, (16,), 0)

        def idx_copy(row, b):
            return pltpu.make_async_copy(src_hbm.at[row], idxb[b], isem.at[b])

        def out_copy(row, b):
            return pltpu.make_async_copy(rowsb[b], out_hbm.at[row], osem.at[b])

        for b in range(KR):
            row0 = b * NW + wid

            @pl.when(row0 < R)
            def _(row0=row0, b=b):
                idx_copy(row0, b).start()

        pltpu.sync_copy(h_hbm, h_v)

        def outer(jj, carry):
            for b in range(KR):
                row = (jj * KR + b) * NW + wid
                prow = row - KR * NW

                @pl.when((jj > 0) & (prow < R))
                def _(prow=prow, b=b):
                    out_copy(prow, b).wait()

                @pl.when(row < R)
                def _(row=row, b=b):
                    idx_copy(row, b).wait()

                    def group(g, carry2):
                        sidx = idxb[b][pl.ds(g * 16, 16)] * HID
                        dbase = (lane + g * 16) * HID
                        for o in range(HID):
                            vals = plsc.load_gather(h_v, [sidx + o])
                            plsc.store_scatter(rowsb[b], [dbase + o], vals)
                        return carry2

                    lax.fori_loop(0, WIN // 16, group, 0)
                    out_copy(row, b).start()
                    nrow = row + KR * NW

                    @pl.when(nrow < R)
                    def _():
                        idx_copy(nrow, b).start()

            return carry

        lax.fori_loop(0, JO, outer, 0)
        for b in range(KR):
            rowl = ((JO - 1) * KR + b) * NW + wid

            @pl.when(rowl < R)
            def _(rowl=rowl, b=b):
                out_copy(rowl, b).wait()

    return k(h_flat, src2d)


def _sc_scatter(msg2d, dst2d, zeros_flat):
    R = msg2d.shape[0]
    NH = zeros_flat.shape[0]
    JMAX = (R + NW - 1) // NW
    JO = (JMAX + KR - 1) // KR
    mesh = plsc.VectorSubcoreMesh(core_axis_name="c", subcore_axis_name="s",
                                  num_cores=NC, num_subcores=NS)

    @functools.partial(
        pl.kernel,
        out_type=jax.ShapeDtypeStruct((NW, NH), F32),
        mesh=mesh,
        compiler_params=pltpu.CompilerParams(needs_layout_passes=False),
        scratch_types=[
            pltpu.VMEM((NH,), F32),
            [pltpu.VMEM((WIN,), jnp.int32) for _ in range(KR)],
            [pltpu.VMEM((WIN * HID,), F32) for _ in range(KR)],
            pltpu.SemaphoreType.DMA((KR,)),
            pltpu.SemaphoreType.DMA((KR,)),
        ],
    )
    def k(msg_hbm, dst_hbm, zero_hbm, out_hbm, agg_v, idxb, updb, isem, usem):
        c_ax = lax.axis_index("c")
        s_ax = lax.axis_index("s")
        wid = s_ax * NC + c_ax
        lane = lax.broadcasted_iota(jnp.int32, (16,), 0)

        def idx_copy(row, b):
            return pltpu.make_async_copy(dst_hbm.at[row], idxb[b], isem.at[b])

        def upd_copy(row, b):
            return pltpu.make_async_copy(msg_hbm.at[row], updb[b], usem.at[b])

        for b in range(KR):
            row0 = b * NW + wid

            @pl.when(row0 < R)
            def _(row0=row0, b=b):
                idx_copy(row0, b).start()
                upd_copy(row0, b).start()

        pltpu.sync_copy(zero_hbm, agg_v)

        def outer(jj, carry):
            for b in range(KR):
                row = (jj * KR + b) * NW + wid

                @pl.when(row < R)
                def _(row=row, b=b):
                    idx_copy(row, b).wait()
                    upd_copy(row, b).wait()

                    def group(g, carry2):
                        didx = idxb[b][pl.ds(g * 16, 16)] * HID
                        ubase = (lane + g * 16) * HID
                        for o in range(HID):
                            vals = plsc.load_gather(updb[b], [ubase + o])
                            plsc.addupdate_scatter(agg_v, [didx + o], vals)
                        return carry2

                    lax.fori_loop(0, WIN // 16, group, 0)
                    nrow = row + KR * NW

                    @pl.when(nrow < R)
                    def _():
                        idx_copy(nrow, b).start()
                        upd_copy(nrow, b).start()

            return carry

        lax.fori_loop(0, JO, outer, 0)
        pltpu.sync_copy(agg_v, out_hbm.at[wid])

    return k(msg2d, dst2d, zeros_flat)


def kernel(x, edge_index, edge_attr, batch, W_proj, b_proj, W_e1, b_e1, W_e2,
           b_e2, W_root, b_conv, W_gru_ih, b_gru_ih, W_gru_hh, b_gru_hh, W_r1,
           b_r1, W_r2, b_r2, W_p, b_p):
    N = x.shape[0]
    E = edge_index.shape[1]
    R = E // WIN

    src2d = edge_index[0].reshape(R, WIN)
    dst2d = edge_index[1].reshape(R, WIN)
    zeros_flat = jnp.zeros((N * HID,), F32)
    batch3d = batch.reshape(N // 2000, 1, 2000)

    r2 = lambda v: v.reshape(1, -1)
    b_proj2 = r2(b_proj)
    b_conv2, b_ih2, b_hh2 = r2(b_conv), r2(b_gru_ih), r2(b_gru_hh)
    br12, br22, bp2 = r2(b_r1), r2(b_r2), r2(b_p)

    DE = edge_attr.shape[1]
    HH = HID * HID
    eye8 = jnp.eye(PK, dtype=F32)
    ii = lax.broadcasted_iota(jnp.int32, (HID, HH), 0)
    cc = lax.broadcasted_iota(jnp.int32, (HID, HH), 1)
    Rm = (cc // HID == ii).astype(F32)
    cc2 = lax.broadcasted_iota(jnp.int32, (HH, HID), 0)
    oo = lax.broadcasted_iota(jnp.int32, (HH, HID), 1)
    Sm = (cc2 % HID == oo).astype(F32)
    W1bd = jnp.kron(eye8, W_e1)
    b1bd = jnp.tile(b_e1, PK).reshape(1, -1)
    W2bd = jnp.kron(eye8, W_e2)
    b2bd = jnp.tile(b_e2, PK).reshape(1, -1)
    Rbd = jnp.kron(eye8, Rm)
    Sbd = jnp.kron(eye8, Sm)
    ea8 = edge_attr.reshape(E // PK, PK * DE)

    h = _tc_proj(x, W_proj, b_proj2)
    G = _sc_gather(h, src2d)
    for step in range(STEPS):
        msg = _tc_msg(ea8, G.reshape(E // PK, PK * HID), W1bd, b1bd, W2bd,
                      b2bd, Rbd, Sbd)
        partials = _sc_scatter(msg.reshape(R, WIN * HID), dst2d, zeros_flat)
        agg = _tc_reduce(partials).reshape(N, HID)
        if step < STEPS - 1:
            h = _tc_gru(agg, h, W_root, b_conv2, W_gru_ih, b_ih2,
                        W_gru_hh, b_hh2)
            G = _sc_gather(h, src2d)
        else:
            out = _tc_gru_readout(agg, h, batch3d, W_root, b_conv2,
                                  W_gru_ih, b_ih2, W_gru_hh, b_hh2,
                                  W_r1, br12, W_r2, br22, W_p, bp2)
    return out
